# Initial kernel scaffold; baseline (speedup 1.0000x reference)
import jax
import jax.numpy as jnp
from jax.experimental import pallas as pl


def kernel(examples, table):
    raise NotImplementedError("write your pallas kernel here")


# speedup vs baseline: 2.3420x; 2.3420x over previous
import functools

import jax
import jax.numpy as jnp
from jax import lax
from jax.experimental import pallas as pl
from jax.experimental.pallas import tpu as pltpu
from jax.experimental.pallas import tpu_sc as plsc

VOCAB = 100000
DIM = 64
BATCH = 4096
HIST = 50
EPS = 1e-5

NC = 2
NS = 16
NW = NC * NS

ROWS = BATCH * HIST
PER_W = ROWS // NW
CHUNK = 128
NCH = PER_W // CHUNK

MAXNORM = 1.0 - EPS
MAXNORM2 = MAXNORM * MAXNORM


def _build():
    mesh = plsc.VectorSubcoreMesh(core_axis_name="c", subcore_axis_name="s")

    @functools.partial(
        pl.kernel,
        mesh=mesh,
        out_type=jax.ShapeDtypeStruct((ROWS, DIM), jnp.float32),
        scratch_types=[
            pltpu.VMEM((NCH, CHUNK), jnp.int32),
            pltpu.VMEM((CHUNK, DIM), jnp.float32),
            pltpu.SemaphoreType.DMA,
        ],
        compiler_params=pltpu.CompilerParams(use_tc_tiling_on_sc=False),
    )
    def body(table_hbm, idx_hbm, out_hbm, idx_v, buf, sem):
        wid = lax.axis_index("s") * NC + lax.axis_index("c")
        pltpu.sync_copy(idx_hbm.at[wid], idx_v)

        lanes = lax.iota(jnp.int32, 16)
        perms = [lanes ^ shift for shift in (8, 4, 2, 1)]

        def chunk_body(c, carry):
            pltpu.async_copy(table_hbm.at[idx_v.at[c]], buf, sem).wait()

            def row_body(r, rcarry):
                v0 = buf[r, pl.ds(0, 16)]
                v1 = buf[r, pl.ds(16, 16)]
                v2 = buf[r, pl.ds(32, 16)]
                v3 = buf[r, pl.ds(48, 16)]
                x = v0 * v0 + v1 * v1 + v2 * v2 + v3 * v3
                for p in perms:
                    x = x + x.at[p].get(mode="promise_in_bounds")
                i = lax.bitcast_convert_type(x, jnp.int32)
                i = jnp.int32(0x5F3759DF) - lax.shift_right_logical(i, 1)
                y = lax.bitcast_convert_type(i, jnp.float32)
                for _ in range(3):
                    y = y * (1.5 - 0.5 * x * y * y)
                scale = jnp.where(
                    x > MAXNORM2,
                    MAXNORM * y,
                    jnp.full((16,), 1.0, dtype=jnp.float32),
                )
                buf[r, pl.ds(0, 16)] = v0 * scale
                buf[r, pl.ds(16, 16)] = v1 * scale
                buf[r, pl.ds(32, 16)] = v2 * scale
                buf[r, pl.ds(48, 16)] = v3 * scale
                return rcarry

            lax.fori_loop(0, CHUNK, row_body, 0)
            pltpu.sync_copy(
                buf, out_hbm.at[pl.ds(wid * PER_W + c * CHUNK, CHUNK)]
            )
            return carry

        lax.fori_loop(0, NCH, chunk_body, 0)

    return body


_sc_lookup = _build()


def kernel(examples, table):
    idx = examples.reshape(NW, NCH, CHUNK)
    out = _sc_lookup(table, idx)
    return out.reshape(BATCH, HIST, DIM)


# speedup vs baseline: 4.3820x; 1.8711x over previous
import functools

import jax
import jax.numpy as jnp
from jax import lax
from jax.experimental import pallas as pl
from jax.experimental.pallas import tpu as pltpu
from jax.experimental.pallas import tpu_sc as plsc

VOCAB = 100000
DIM = 64
BATCH = 4096
HIST = 50
EPS = 1e-5

NC = 2
NS = 16
NW = NC * NS

ROWS = BATCH * HIST
PER_W = ROWS // NW
CHUNK = 128
NCH = PER_W // CHUNK
NB = 2

MAXNORM = 1.0 - EPS
MAXNORM2 = MAXNORM * MAXNORM


def _build():
    mesh = plsc.VectorSubcoreMesh(core_axis_name="c", subcore_axis_name="s")

    @functools.partial(
        pl.kernel,
        mesh=mesh,
        out_type=jax.ShapeDtypeStruct((ROWS, DIM), jnp.float32),
        scratch_types=[
            pltpu.VMEM((NCH, CHUNK), jnp.int32),
            pltpu.VMEM((NB, CHUNK, DIM), jnp.float32),
            pltpu.VMEM((NB, CHUNK, DIM), jnp.float32),
            pltpu.SemaphoreType.DMA,
            pltpu.SemaphoreType.DMA,
            pltpu.SemaphoreType.DMA,
            pltpu.SemaphoreType.DMA,
        ],
        compiler_params=pltpu.CompilerParams(use_tc_tiling_on_sc=False),
    )
    def body(table_hbm, idx_hbm, out_hbm, idx_v, inb, outb, g0, g1, s0, s1):
        wid = lax.axis_index("s") * NC + lax.axis_index("c")
        out_base = wid * PER_W
        pltpu.sync_copy(idx_hbm.at[wid], idx_v)
        gsems = [g0, g1]
        ssems = [s0, s1]

        lanes = lax.iota(jnp.int32, 16)
        perms = [lanes ^ shift for shift in (8, 4, 2, 1)]

        def gather(c, b):
            return pltpu.make_async_copy(
                table_hbm.at[idx_v.at[c]], inb.at[b], gsems[b]
            )

        def scatter(c, b):
            return pltpu.make_async_copy(
                outb.at[b], out_hbm.at[pl.ds(out_base + c * CHUNK, CHUNK)], ssems[b]
            )

        for b in range(NB):
            gather(jnp.int32(b), b).start()

        def outer(g, carry):
            for b in range(NB):
                c = g * NB + b
                gather(c, b).wait()

                @pl.when(g > 0)
                def _():
                    scatter(jnp.int32(0), b).wait()

                src = inb.at[b]
                dst = outb.at[b]

                @plsc.parallel_loop(0, CHUNK, unroll=4)
                def _(r):
                    v0 = src[r, pl.ds(0, 16)]
                    v1 = src[r, pl.ds(16, 16)]
                    v2 = src[r, pl.ds(32, 16)]
                    v3 = src[r, pl.ds(48, 16)]
                    x = v0 * v0 + v1 * v1 + v2 * v2 + v3 * v3
                    for p in perms:
                        x = x + x.at[p].get(mode="promise_in_bounds")
                    i = lax.bitcast_convert_type(x, jnp.int32)
                    i = jnp.int32(0x5F3759DF) - lax.shift_right_logical(i, 1)
                    y = lax.bitcast_convert_type(i, jnp.float32)
                    for _ in range(3):
                        y = y * (1.5 - 0.5 * x * y * y)
                    scale = jnp.where(
                        x > MAXNORM2,
                        MAXNORM * y,
                        jnp.full((16,), 1.0, dtype=jnp.float32),
                    )
                    dst[r, pl.ds(0, 16)] = v0 * scale
                    dst[r, pl.ds(16, 16)] = v1 * scale
                    dst[r, pl.ds(32, 16)] = v2 * scale
                    dst[r, pl.ds(48, 16)] = v3 * scale

                scatter(c, b).start()

                @pl.when(c + NB < NCH)
                def _():
                    gather(c + NB, b).start()

            return carry

        lax.fori_loop(0, NCH // NB, outer, 0)
        for b in range(NB):
            scatter(jnp.int32(0), b).wait()

    return body


_sc_lookup = _build()


def kernel(examples, table):
    idx = examples.reshape(NW, NCH, CHUNK)
    out = _sc_lookup(table, idx)
    return out.reshape(BATCH, HIST, DIM)
